# baseline (speedup 1.0000x reference)
import functools

import jax
import jax.numpy as jnp
from jax import lax
from jax.experimental import pallas as pl
from jax.experimental.pallas import tpu as pltpu
from jax.experimental.pallas import tpu_sc as plsc

H = 256
W = 256
D = 768
LANES = 16
NC = 2
NS = 16
NW = NC * NS
RPW = H // NW
JC = 32
NJ = W // JC
LG = D // LANES
NT = NJ * RPW

_mesh = plsc.VectorSubcoreMesh(core_axis_name="c", subcore_axis_name="s")


@functools.partial(
    pl.kernel,
    mesh=_mesh,
    out_type=jax.ShapeDtypeStruct((H * W, D), jnp.float32),
    scratch_types=[
        pltpu.VMEM((RPW,), jnp.int32),
        pltpu.VMEM((2 * JC,), jnp.int32),
        pltpu.VMEM((RPW, D), jnp.float32),
        pltpu.VMEM((2 * JC, D), jnp.float32),
        pltpu.VMEM((2 * JC, D), jnp.float32),
        pltpu.SemaphoreType.DMA,
        pltpu.SemaphoreType.DMA,
        pltpu.SemaphoreType.DMA,
        pltpu.SemaphoreType.DMA,
        pltpu.SemaphoreType.DMA,
    ],
)
def _sc_embed(rows_hbm, cols_hbm, row_table, col_table, out_hbm,
              ridx_v, cidx_v, rowe_v, cole_v, outb_v,
              sem_row, sem_c0, sem_c1, sem_o0, sem_o1):
    wid = lax.axis_index("s") * NC + lax.axis_index("c")
    rbase = wid * RPW

    pltpu.sync_copy(rows_hbm.at[pl.ds(rbase, RPW)], ridx_v)
    row_cp = pltpu.make_async_copy(
        row_table.at[ridx_v], rowe_v, sem_row)
    row_cp.start()

    pltpu.sync_copy(cols_hbm.at[pl.ds(0, JC)], cidx_v.at[pl.ds(0, JC)])
    pltpu.make_async_copy(
        col_table.at[cidx_v.at[pl.ds(0, JC)]],
        cole_v.at[pl.ds(0, JC)], sem_c0).start()

    row_cp.wait()

    def _col_wait(half):
        @pl.when(half == 0)
        def _():
            pltpu.make_async_copy(
                col_table.at[cidx_v.at[pl.ds(0, JC)]],
                cole_v.at[pl.ds(0, JC)], sem_c0).wait()

        @pl.when(half == 1)
        def _():
            pltpu.make_async_copy(
                col_table.at[cidx_v.at[pl.ds(JC, JC)]],
                cole_v.at[pl.ds(JC, JC)], sem_c1).wait()

    def _col_fetch(half, cj):
        cb = half * JC
        pltpu.sync_copy(cols_hbm.at[pl.ds(cj * JC, JC)],
                        cidx_v.at[pl.ds(cb, JC)])

        @pl.when(half == 0)
        def _():
            pltpu.make_async_copy(
                col_table.at[cidx_v.at[pl.ds(0, JC)]],
                cole_v.at[pl.ds(0, JC)], sem_c0).start()

        @pl.when(half == 1)
        def _():
            pltpu.make_async_copy(
                col_table.at[cidx_v.at[pl.ds(JC, JC)]],
                cole_v.at[pl.ds(JC, JC)], sem_c1).start()

    def _out_sem_op(half, dst, is_wait):
        @pl.when(half == 0)
        def _():
            cp = pltpu.make_async_copy(outb_v.at[pl.ds(0, JC)], dst, sem_o0)
            _ = cp.wait() if is_wait else cp.start()

        @pl.when(half == 1)
        def _():
            cp = pltpu.make_async_copy(outb_v.at[pl.ds(JC, JC)], dst, sem_o1)
            _ = cp.wait() if is_wait else cp.start()

    def block(t, _):
        cj = t // RPW
        il = t % RPW
        p = t % 2
        bb = p * JC
        cp_half = cj % 2
        cb = cp_half * JC

        @pl.when(il == 0)
        def _():
            _col_wait(cp_half)

            @pl.when(cj < NJ - 1)
            def _():
                _col_fetch(1 - cp_half, cj + 1)

        @pl.when(t >= 2)
        def _():
            _out_sem_op(p, out_hbm.at[pl.ds(0, JC)], True)

        rvs = [rowe_v[il, pl.ds(g * LANES, LANES)] for g in range(LG)]

        def r_body(r, _):
            for g in range(LG):
                sl = pl.ds(g * LANES, LANES)
                outb_v[bb + r, sl] = cole_v[cb + r, sl] + rvs[g]
            return 0

        lax.fori_loop(0, JC, r_body, 0)

        out_start = (rbase + il) * W + cj * JC
        _out_sem_op(p, out_hbm.at[pl.ds(out_start, JC)], False)
        return 0

    lax.fori_loop(0, NT, block, 0)

    _out_sem_op(0, out_hbm.at[pl.ds(0, JC)], True)
    _out_sem_op(1, out_hbm.at[pl.ds(0, JC)], True)


def kernel(grid_size, row_table, col_table):
    gh = jnp.asarray(grid_size[0], jnp.int32)
    gw = jnp.asarray(grid_size[1], jnp.int32)
    rows = jnp.minimum(jnp.arange(H, dtype=jnp.int32), gh - 1)
    cols = jnp.minimum(jnp.arange(W, dtype=jnp.int32), gw - 1)
    return _sc_embed(rows, cols, row_table, col_table)


# speedup vs baseline: 3.7399x; 3.7399x over previous
import functools

import jax
import jax.numpy as jnp
from jax import lax
from jax.experimental import pallas as pl
from jax.experimental.pallas import tpu as pltpu
from jax.experimental.pallas import tpu_sc as plsc

H = 256
W = 256
D = 768
LANES = 16
NC = 2
NS = 16
NW = NC * NS
RPW = H // NW
JC = 32
NJ = W // JC
LG = D // LANES

_mesh = plsc.VectorSubcoreMesh(core_axis_name="c", subcore_axis_name="s")


@functools.partial(
    pl.kernel,
    mesh=_mesh,
    out_type=jax.ShapeDtypeStruct((H * W, D), jnp.float32),
    scratch_types=[
        pltpu.VMEM((RPW,), jnp.int32),
        pltpu.VMEM((JC,), jnp.int32),
        pltpu.VMEM((RPW, D), jnp.float32),
        pltpu.VMEM((JC, D), jnp.float32),
        pltpu.VMEM((JC, D), jnp.float32),
        pltpu.VMEM((JC, D), jnp.float32),
        pltpu.SemaphoreType.DMA,
        pltpu.SemaphoreType.DMA,
        pltpu.SemaphoreType.DMA,
    ],
)
def _sc_embed(rows_hbm, cols_hbm, row_table, col_table, out_hbm,
              ridx_v, cidx_v, rowe_v, cole_v, outb0_v, outb1_v,
              sem_g, sem_o0, sem_o1):
    wid = lax.axis_index("s") * NC + lax.axis_index("c")
    rbase = wid * RPW
    pltpu.sync_copy(rows_hbm.at[pl.ds(rbase, RPW)], ridx_v)
    pltpu.async_copy(row_table.at[ridx_v], rowe_v, sem_g).wait()

    def chunk_body(cj, _):
        j0 = cj * JC
        pltpu.sync_copy(cols_hbm.at[pl.ds(j0, JC)], cidx_v)
        pltpu.async_copy(col_table.at[cidx_v], cole_v, sem_g).wait()

        def pair_body(tp, _):
            for outb_v, sem_o, b in ((outb0_v, sem_o0, 0), (outb1_v, sem_o1, 1)):
                il = tp * 2 + b

                @pl.when((cj > 0) | (tp > 0))
                def _():
                    pltpu.make_async_copy(
                        outb_v, out_hbm.at[pl.ds(0, JC)], sem_o).wait()

                rvs = [rowe_v[il, pl.ds(g * LANES, LANES)] for g in range(LG)]

                def r_body(r, _):
                    for g in range(LG):
                        sl = pl.ds(g * LANES, LANES)
                        outb_v[r, sl] = cole_v[r, sl] + rvs[g]
                    return 0

                lax.fori_loop(0, JC, r_body, 0)
                out_start = (rbase + il) * W + j0
                pltpu.make_async_copy(
                    outb_v, out_hbm.at[pl.ds(out_start, JC)], sem_o).start()
            return 0

        lax.fori_loop(0, RPW // 2, pair_body, 0)
        return 0

    lax.fori_loop(0, NJ, chunk_body, 0)

    pltpu.make_async_copy(outb0_v, out_hbm.at[pl.ds(0, JC)], sem_o0).wait()
    pltpu.make_async_copy(outb1_v, out_hbm.at[pl.ds(0, JC)], sem_o1).wait()


def kernel(grid_size, row_table, col_table):
    gh = jnp.asarray(grid_size[0], jnp.int32)
    gw = jnp.asarray(grid_size[1], jnp.int32)
    rows = jnp.minimum(jnp.arange(H, dtype=jnp.int32), gh - 1)
    cols = jnp.minimum(jnp.arange(W, dtype=jnp.int32), gw - 1)
    return _sc_embed(rows, cols, row_table, col_table)


# speedup vs baseline: 4.1395x; 1.1069x over previous
import functools

import jax
import jax.numpy as jnp
from jax import lax
from jax.experimental import pallas as pl
from jax.experimental.pallas import tpu as pltpu
from jax.experimental.pallas import tpu_sc as plsc

H = 256
W = 256
D = 768
LANES = 16
NC = 2
NS = 16
NW = NC * NS
RPW = H // NW
JC = 32
NJ = W // JC
LG = D // LANES

_mesh = plsc.VectorSubcoreMesh(core_axis_name="c", subcore_axis_name="s")


@functools.partial(
    pl.kernel,
    mesh=_mesh,
    out_type=jax.ShapeDtypeStruct((H * W, D), jnp.float32),
    scratch_types=[
        pltpu.VMEM((RPW,), jnp.int32),
        pltpu.VMEM((JC,), jnp.int32),
        pltpu.VMEM((JC,), jnp.int32),
        pltpu.VMEM((RPW, D), jnp.float32),
        pltpu.VMEM((JC, D), jnp.float32),
        pltpu.VMEM((JC, D), jnp.float32),
        pltpu.VMEM((JC, D), jnp.float32),
        pltpu.VMEM((JC, D), jnp.float32),
        pltpu.SemaphoreType.DMA,
        pltpu.SemaphoreType.DMA,
        pltpu.SemaphoreType.DMA,
        pltpu.SemaphoreType.DMA,
    ],
)
def _sc_embed(rows_hbm, cols_hbm, row_table, col_table, out_hbm,
              ridx_v, cidx0_v, cidx1_v, rowe_v, cole0_v, cole1_v,
              outb0_v, outb1_v, sem_row, sem_c, sem_o0, sem_o1):
    wid = lax.axis_index("s") * NC + lax.axis_index("c")
    rbase = wid * RPW

    pltpu.sync_copy(rows_hbm.at[pl.ds(rbase, RPW)], ridx_v)
    row_cp = pltpu.make_async_copy(row_table.at[ridx_v], rowe_v, sem_row)
    row_cp.start()

    def col_gather(cidx_v, cole_v, cj):
        pltpu.sync_copy(cols_hbm.at[pl.ds(cj * JC, JC)], cidx_v)
        pltpu.make_async_copy(col_table.at[cidx_v], cole_v, sem_c).start()

    col_gather(cidx0_v, cole0_v, 0)
    row_cp.wait()

    halves = ((cidx0_v, cole0_v), (cidx1_v, cole1_v))
    bufs = ((outb0_v, sem_o0), (outb1_v, sem_o1))

    def chunk_pair_body(cjp, _):
        for half, (cidx_v, cole_v) in enumerate(halves):
            cj = cjp * 2 + half
            pltpu.make_async_copy(
                col_table.at[cidx_v], cole_v, sem_c).wait()
            n_cidx, n_cole = halves[1 - half]

            @pl.when(cj < NJ - 1)
            def _():
                col_gather(n_cidx, n_cole, cj + 1)

            def pair_body(tp, _):
                for b, (outb_v, sem_o) in enumerate(bufs):
                    il = tp * 2 + b
                    first_use = (cj == 0) & (tp == 0) if half == 0 else None

                    def wait_out():
                        pltpu.make_async_copy(
                            outb_v, out_hbm.at[pl.ds(0, JC)], sem_o).wait()

                    if half == 0:
                        @pl.when((cjp > 0) | (tp > 0))
                        def _():
                            wait_out()
                    else:
                        wait_out()

                    rvs = [rowe_v[il, pl.ds(g * LANES, LANES)]
                           for g in range(LG)]

                    def r_body(r, _):
                        for g in range(LG):
                            sl = pl.ds(g * LANES, LANES)
                            outb_v[r, sl] = cole_v[r, sl] + rvs[g]
                        return 0

                    lax.fori_loop(0, JC, r_body, 0)
                    out_start = (rbase + il) * W + cj * JC
                    pltpu.make_async_copy(
                        outb_v, out_hbm.at[pl.ds(out_start, JC)],
                        sem_o).start()
                return 0

            lax.fori_loop(0, RPW // 2, pair_body, 0)
        return 0

    lax.fori_loop(0, NJ // 2, chunk_pair_body, 0)

    pltpu.make_async_copy(outb0_v, out_hbm.at[pl.ds(0, JC)], sem_o0).wait()
    pltpu.make_async_copy(outb1_v, out_hbm.at[pl.ds(0, JC)], sem_o1).wait()


def kernel(grid_size, row_table, col_table):
    gh = jnp.asarray(grid_size[0], jnp.int32)
    gw = jnp.asarray(grid_size[1], jnp.int32)
    rows = jnp.minimum(jnp.arange(H, dtype=jnp.int32), gh - 1)
    cols = jnp.minimum(jnp.arange(W, dtype=jnp.int32), gw - 1)
    return _sc_embed(rows, cols, row_table, col_table)
